# Initial kernel scaffold; baseline (speedup 1.0000x reference)
import jax
import jax.numpy as jnp
from jax.experimental import pallas as pl


def kernel(input, target, table):
    raise NotImplementedError("write your pallas kernel here")


# speedup vs baseline: 2.0799x; 2.0799x over previous
import functools

import jax
import jax.numpy as jnp
from jax import lax
from jax.experimental import pallas as pl
from jax.experimental.pallas import tpu as pltpu
from jax.experimental.pallas import tpu_sc as plsc

V = 65
VP = 80
B, T = 64, 2048
N = B * T

NC, NS, L = 2, 16, 16
NW = NC * NS
TPW = N // NW
CH = 1024
NCH = TPW // CH


def _nll_table_body(table_ref, nll_ref):
    x = table_ref[...]
    m = jnp.max(x, axis=-1, keepdims=True)
    lse = m + jnp.log(jnp.sum(jnp.exp(x - m), axis=-1, keepdims=True))
    nll_ref[...] = lse - x


_nll_table = pl.pallas_call(
    _nll_table_body,
    out_shape=jax.ShapeDtypeStruct((V, V), jnp.float32),
)


_sc_mesh = plsc.VectorSubcoreMesh(
    core_axis_name="c", subcore_axis_name="s", num_cores=NC, num_subcores=NS
)


@functools.partial(
    pl.kernel,
    out_type=(
        jax.ShapeDtypeStruct((N * V,), jnp.float32),
        jax.ShapeDtypeStruct((NW, L), jnp.float32),
    ),
    mesh=_sc_mesh,
    compiler_params=pltpu.CompilerParams(
        needs_layout_passes=False, use_tc_tiling_on_sc=False
    ),
    scratch_types=[
        pltpu.VMEM((CH,), jnp.int32),
        pltpu.VMEM((CH,), jnp.int32),
        pltpu.VMEM((CH * V + L,), jnp.float32),
        pltpu.VMEM((V * VP,), jnp.float32),
        pltpu.VMEM((V * V,), jnp.float32),
        pltpu.VMEM((L,), jnp.float32),
    ],
)
def _sc_body(inp_hbm, tgt_hbm, tabp_hbm, nll_hbm, out_hbm, part_hbm,
             idx_v, tgt_v, rows_v, tab_v, nll_v, part_v):
    wid = lax.axis_index("s") * NC + lax.axis_index("c")
    base = wid * TPW

    pltpu.sync_copy(tabp_hbm, tab_v)
    pltpu.sync_copy(nll_hbm, nll_v)

    total = jnp.zeros((L,), jnp.float32)
    for c in range(NCH):
        off = base + c * CH
        pltpu.sync_copy(inp_hbm.at[pl.ds(off, CH)], idx_v)
        pltpu.sync_copy(tgt_hbm.at[pl.ds(off, CH)], tgt_v)

        def group(g, acc):
            iv = idx_v[pl.ds(g * L, L)]
            tv = tgt_v[pl.ds(g * L, L)]
            acc = acc + plsc.load_gather(nll_v, [iv * V + tv])
            for j in range(L):
                src = iv[j] * VP
                dst = (g * L + j) * V
                for k in range(5):
                    rows_v[pl.ds(dst + k * L, L)] = (
                        tab_v[pl.ds(src + k * L, L)]
                    )
            return acc

        total = lax.fori_loop(0, CH // L, group, total)
        pltpu.sync_copy(
            rows_v.at[pl.ds(0, CH * V)], out_hbm.at[pl.ds(off * V, CH * V)]
        )

    part_v[...] = total * (1.0 / N)
    pltpu.sync_copy(part_v, part_hbm.at[wid])


def kernel(input, target, table):
    nll = _nll_table(table).reshape(V * V)
    table_pad = jnp.pad(table, ((0, 0), (0, VP - V))).reshape(V * VP)
    logits_flat, parts = _sc_body(
        input.reshape(N), target.reshape(N), table_pad, nll
    )
    return logits_flat.reshape(B, T, V), jnp.sum(parts)


# speedup vs baseline: 2.1997x; 1.0576x over previous
import functools

import jax
import jax.numpy as jnp
from jax import lax
from jax.experimental import pallas as pl
from jax.experimental.pallas import tpu as pltpu
from jax.experimental.pallas import tpu_sc as plsc

V = 65
B, T = 64, 2048
N = B * T

NC, NS, L = 2, 16, 16
NW = NC * NS
TPW = N // NW
CH = 512
NCH = TPW // CH


def _nll_table_body(table_ref, nll_ref):
    x = table_ref[...]
    m = jnp.max(x, axis=-1, keepdims=True)
    lse = m + jnp.log(jnp.sum(jnp.exp(x - m), axis=-1, keepdims=True))
    nll_ref[...] = lse - x


_nll_table = pl.pallas_call(
    _nll_table_body,
    out_shape=jax.ShapeDtypeStruct((V, V), jnp.float32),
)


_sc_mesh = plsc.VectorSubcoreMesh(
    core_axis_name="c", subcore_axis_name="s", num_cores=NC, num_subcores=NS
)


@functools.partial(
    pl.kernel,
    out_type=(
        jax.ShapeDtypeStruct((N * V,), jnp.float32),
        jax.ShapeDtypeStruct((NW, L), jnp.float32),
    ),
    mesh=_sc_mesh,
    compiler_params=pltpu.CompilerParams(
        needs_layout_passes=False, use_tc_tiling_on_sc=False
    ),
    scratch_types=[
        pltpu.VMEM((TPW,), jnp.int32),
        pltpu.VMEM((TPW,), jnp.int32),
        pltpu.VMEM((CH * V,), jnp.float32),
        pltpu.VMEM((CH * V,), jnp.float32),
        pltpu.VMEM((V * V,), jnp.float32),
        pltpu.VMEM((V * V,), jnp.float32),
        pltpu.VMEM((L,), jnp.float32),
        pltpu.SemaphoreType.DMA,
        pltpu.SemaphoreType.DMA,
    ],
)
def _sc_body(inp_hbm, tgt_hbm, tabp_hbm, nll_hbm, out_hbm, part_hbm,
             idx_v, tgt_v, rows_a, rows_b, tab_v, nll_v, part_v,
             sem_a, sem_b):
    wid = lax.axis_index("s") * NC + lax.axis_index("c")
    base = wid * TPW

    pltpu.sync_copy(tabp_hbm, tab_v)
    pltpu.sync_copy(nll_hbm, nll_v)
    pltpu.sync_copy(inp_hbm.at[pl.ds(base, TPW)], idx_v)
    pltpu.sync_copy(tgt_hbm.at[pl.ds(base, TPW)], tgt_v)

    def loss_group(g, acc):
        iv = idx_v[pl.ds(g * L, L)]
        tv = tgt_v[pl.ds(g * L, L)]
        return acc + plsc.load_gather(nll_v, [iv * V + tv])

    total = lax.fori_loop(0, TPW // L, loss_group, jnp.zeros((L,), jnp.float32))
    part_v[...] = total * (1.0 / N)
    pltpu.sync_copy(part_v, part_hbm.at[wid])

    bufs = (rows_a, rows_b)
    sems = (sem_a, sem_b)
    pending = [None, None]
    for c in range(NCH):
        slot = c % 2
        buf = bufs[slot]
        if pending[slot] is not None:
            pending[slot].wait()

        lane = lax.iota(jnp.int32, L)

        def build(g, carry):
            iv = idx_v[pl.ds(c * CH + g * L, L)]
            srcb = iv * V
            dstb = (g * L + lane) * V
            for v in range(V):
                vals = plsc.load_gather(tab_v, [srcb + v])
                plsc.store_scatter(buf, [dstb + v], vals)
            return carry

        lax.fori_loop(0, CH // L, build, 0)
        off = (base + c * CH) * V
        pending[slot] = pltpu.async_copy(
            buf, out_hbm.at[pl.ds(off, CH * V)], sems[slot],
        )
    pending[0].wait()
    pending[1].wait()


def kernel(input, target, table):
    nll = _nll_table(table).reshape(V * V)
    logits_flat, parts = _sc_body(
        input.reshape(N), target.reshape(N), table.reshape(V * V), nll
    )
    return logits_flat.reshape(B, T, V), jnp.sum(parts)
